# Initial kernel scaffold; baseline (speedup 1.0000x reference)
import jax
import jax.numpy as jnp
from jax.experimental import pallas as pl


def kernel(x, edge_index, W1, b1, W2, b2, fcW, fcb):
    raise NotImplementedError("write your pallas kernel here")


# speedup vs baseline: 17.2112x; 17.2112x over previous
import functools

import jax
import jax.numpy as jnp
from jax import lax
from jax.experimental import pallas as pl
from jax.experimental.pallas import tpu as pltpu
from jax.experimental.pallas import tpu_sc as plsc

NC = 2
NS = 16
NW = NC * NS
CHUNK = 128
BLK = 512
DEG_D = 16


@functools.lru_cache(maxsize=None)
def _make_agg(n_pad: int, d: int, cpw: int):
    rpt = n_pad // NS
    mesh = plsc.VectorSubcoreMesh(core_axis_name="c", subcore_axis_name="s")

    @functools.partial(
        pl.kernel,
        out_type=jax.ShapeDtypeStruct((NC, n_pad, d), jnp.float32),
        mesh=mesh,
        scratch_types=[
            pltpu.VMEM((cpw, CHUNK), jnp.int32),
            pltpu.VMEM((cpw, CHUNK), jnp.int32),
            pltpu.VMEM((CHUNK, d), jnp.float32),
            pltpu.VMEM((CHUNK, d), jnp.float32),
            pltpu.SemaphoreType.DMA,
            pltpu.SemaphoreType.DMA,
            pltpu.VMEM_SHARED((n_pad, d), jnp.float32),
        ],
        compiler_params=pltpu.CompilerParams(use_tc_tiling_on_sc=False),
    )
    def agg(tbl, srcs, dsts, out, src_v, dst_v, buf0, buf1, sem0, sem1, acc):
        c = lax.axis_index("c")
        s = lax.axis_index("s")
        w = c * NS + s
        pltpu.sync_copy(srcs.at[pl.ds(w * cpw, cpw)], src_v)
        pltpu.sync_copy(dsts.at[pl.ds(w * cpw, cpw)], dst_v)
        r0 = s * rpt
        pltpu.sync_copy(tbl.at[pl.ds(r0, rpt)], acc.at[pl.ds(r0, rpt)])
        plsc.subcore_barrier()

        bufs = (buf0, buf1)
        sems = (sem0, sem1)
        for b in range(2):
            pltpu.async_copy(tbl.at[src_v.at[b]], bufs[b], sems[b])

        def halfstep(r, b):
            buf, sem = bufs[b], sems[b]
            pltpu.make_async_copy(tbl.at[src_v.at[r]], buf, sem).wait()
            pltpu.sync_copy(buf, acc.at[dst_v.at[r]], add=True)

            @pl.when(r + 2 < cpw)
            def _():
                pltpu.async_copy(tbl.at[src_v.at[r + 2]], buf, sem)

        def body(g, carry):
            halfstep(2 * g, 0)
            halfstep(2 * g + 1, 1)
            return carry

        lax.fori_loop(0, cpw // 2, body, 0)
        if cpw % 2:
            halfstep(cpw - 1, 0)

        plsc.subcore_barrier()
        pltpu.sync_copy(acc.at[pl.ds(r0, rpt)], out.at[c, pl.ds(r0, rpt)])

    return agg


@functools.lru_cache(maxsize=None)
def _make_deg(n_pad: int, cpw: int):
    rpt = n_pad // NS
    mesh = plsc.VectorSubcoreMesh(core_axis_name="c", subcore_axis_name="s")

    @functools.partial(
        pl.kernel,
        out_type=jax.ShapeDtypeStruct((NC, n_pad, DEG_D), jnp.float32),
        mesh=mesh,
        scratch_types=[
            pltpu.VMEM((cpw, CHUNK), jnp.int32),
            pltpu.VMEM((CHUNK, DEG_D), jnp.float32),
            pltpu.VMEM_SHARED((n_pad, DEG_D), jnp.float32),
        ],
        compiler_params=pltpu.CompilerParams(use_tc_tiling_on_sc=False),
    )
    def deg(ones_hbm, zeros_hbm, dsts, out, dst_v, ones_v, acc):
        c = lax.axis_index("c")
        s = lax.axis_index("s")
        w = c * NS + s
        pltpu.sync_copy(dsts.at[pl.ds(w * cpw, cpw)], dst_v)
        pltpu.sync_copy(ones_hbm, ones_v)
        r0 = s * rpt
        pltpu.sync_copy(zeros_hbm, acc.at[pl.ds(r0, rpt)])
        plsc.subcore_barrier()

        def body(r, carry):
            pltpu.sync_copy(ones_v, acc.at[dst_v.at[r]], add=True)
            return carry

        lax.fori_loop(0, cpw, body, 0)

        plsc.subcore_barrier()
        pltpu.sync_copy(acc.at[pl.ds(r0, rpt)], out.at[c, pl.ds(r0, rpt)])

    return deg


def _mm1_body(x_ref, w_ref, d0_ref, d1_ref, h_ref, dinv_ref):
    deg = d0_ref[...] + d1_ref[...] + 1.0
    dinv = lax.rsqrt(deg)
    dinv_ref[...] = dinv
    h_ref[...] = jnp.dot(x_ref[...], w_ref[...],
                         preferred_element_type=jnp.float32) * dinv


def _mm2_body(a0_ref, a1_ref, hs_ref, dinv_ref, b_ref, w_ref, out_ref):
    dinv = dinv_ref[...]
    h = dinv * (a0_ref[...] + a1_ref[...] - hs_ref[...]) + b_ref[...]
    h = jnp.maximum(h, 0.0)
    out_ref[...] = jnp.dot(h, w_ref[...],
                           preferred_element_type=jnp.float32) * dinv


def _mm3_body(a0_ref, a1_ref, hs_ref, dinv_ref, b_ref, fcw_ref, fcb_ref,
              emb_ref, log_ref):
    dinv = dinv_ref[...]
    h = dinv * (a0_ref[...] + a1_ref[...] - hs_ref[...]) + b_ref[...]
    h = jnp.maximum(h, 0.0)
    emb_ref[...] = h
    log_ref[...] = jnp.dot(h, fcw_ref[...],
                           preferred_element_type=jnp.float32) + fcb_ref[...]


def _row_spec(d):
    return pl.BlockSpec((BLK, d), lambda i: (i, 0))


def _full_spec(shape):
    return pl.BlockSpec(shape, lambda i: (0,) * len(shape))


def kernel(x, edge_index, W1, b1, W2, b2, fcW, fcb):
    n, in_dim = x.shape
    h1d = W1.shape[1]
    h2d = W2.shape[1]
    od = fcW.shape[1]

    src = edge_index[0].astype(jnp.int32)
    dst = edge_index[1].astype(jnp.int32)
    e = src.shape[0]

    n_pad = ((n + 1 + BLK - 1) // BLK) * BLK
    cpw = -(-(-(-e // CHUNK)) // NW)
    cpw = ((cpw + 7) // 8) * 8
    e_pad = cpw * NW * CHUNK

    src_p = jnp.concatenate(
        [src, jnp.zeros((e_pad - e,), jnp.int32)]).reshape(-1, CHUNK)
    dst_p = jnp.concatenate(
        [dst, jnp.full((e_pad - e,), n, jnp.int32)]).reshape(-1, CHUNK)
    x_p = jnp.pad(x, ((0, n_pad - n), (0, 0)))

    ones_rows = jnp.ones((CHUNK, DEG_D), jnp.float32)
    zeros_stripe = jnp.zeros((n_pad // NS, DEG_D), jnp.float32)

    deg_out = _make_deg(n_pad, cpw)(ones_rows, zeros_stripe, dst_p)
    d0 = deg_out[0, :, 0:1]
    d1 = deg_out[1, :, 0:1]

    grid = (n_pad // BLK,)

    h1s, dinv = pl.pallas_call(
        _mm1_body,
        grid=grid,
        in_specs=[_row_spec(in_dim), _full_spec((in_dim, h1d)),
                  _row_spec(1), _row_spec(1)],
        out_specs=[_row_spec(h1d), _row_spec(1)],
        out_shape=[jax.ShapeDtypeStruct((n_pad, h1d), jnp.float32),
                   jax.ShapeDtypeStruct((n_pad, 1), jnp.float32)],
    )(x_p, W1, d0, d1)

    agg1 = _make_agg(n_pad, h1d, cpw)(h1s, src_p, dst_p)

    h2s = pl.pallas_call(
        _mm2_body,
        grid=grid,
        in_specs=[_row_spec(h1d), _row_spec(h1d), _row_spec(h1d), _row_spec(1),
                  _full_spec((1, h1d)), _full_spec((h1d, h2d))],
        out_specs=_row_spec(h2d),
        out_shape=jax.ShapeDtypeStruct((n_pad, h2d), jnp.float32),
    )(agg1[0], agg1[1], h1s, dinv, b1.reshape(1, -1), W2)

    agg2 = _make_agg(n_pad, h2d, cpw)(h2s, src_p, dst_p)

    emb, logits = pl.pallas_call(
        _mm3_body,
        grid=grid,
        in_specs=[_row_spec(h2d), _row_spec(h2d), _row_spec(h2d), _row_spec(1),
                  _full_spec((1, h2d)), _full_spec((h2d, od)),
                  _full_spec((1, od))],
        out_specs=[_row_spec(h2d), _row_spec(od)],
        out_shape=[jax.ShapeDtypeStruct((n_pad, h2d), jnp.float32),
                   jax.ShapeDtypeStruct((n_pad, od), jnp.float32)],
    )(agg2[0], agg2[1], h2s, dinv, b2.reshape(1, -1), fcW, fcb.reshape(1, -1))

    return emb[:n], logits[:n]


# speedup vs baseline: 32.0828x; 1.8641x over previous
import functools

import jax
import jax.numpy as jnp
from jax import lax
from jax.experimental import pallas as pl
from jax.experimental.pallas import tpu as pltpu
from jax.experimental.pallas import tpu_sc as plsc

NC = 2
NS = 16
NW = NC * NS
CHUNK = 128
BLK = 512
DEG_D = 16


@functools.lru_cache(maxsize=None)
def _make_agg(n_pad: int, d: int, cpw: int):
    rpt = n_pad // NS
    mesh = plsc.VectorSubcoreMesh(core_axis_name="c", subcore_axis_name="s")

    @functools.partial(
        pl.kernel,
        out_type=jax.ShapeDtypeStruct((NC, n_pad, d), jnp.float32),
        mesh=mesh,
        scratch_types=[
            pltpu.VMEM((cpw, CHUNK), jnp.int32),
            pltpu.VMEM((cpw, CHUNK), jnp.int32),
            pltpu.VMEM((CHUNK, d), jnp.float32),
            pltpu.VMEM((CHUNK, d), jnp.float32),
            pltpu.SemaphoreType.DMA,
            pltpu.SemaphoreType.DMA,
            pltpu.VMEM_SHARED((n_pad, d), jnp.float32),
            pltpu.VMEM_SHARED((n_pad, d), jnp.float32),
        ],
        compiler_params=pltpu.CompilerParams(use_tc_tiling_on_sc=False),
    )
    def agg(tbl, srcs, dsts, out, src_v, dst_v, buf0, buf1, sem0, sem1, acc,
            tbl_s):
        c = lax.axis_index("c")
        s = lax.axis_index("s")
        w = c * NS + s
        pltpu.sync_copy(srcs.at[pl.ds(w * cpw, cpw)], src_v)
        pltpu.sync_copy(dsts.at[pl.ds(w * cpw, cpw)], dst_v)
        r0 = s * rpt
        pltpu.sync_copy(tbl.at[pl.ds(r0, rpt)], tbl_s.at[pl.ds(r0, rpt)])
        pltpu.sync_copy(tbl.at[pl.ds(r0, rpt)], acc.at[pl.ds(r0, rpt)])
        plsc.subcore_barrier()

        bufs = (buf0, buf1)
        sems = (sem0, sem1)
        for b in range(2):
            pltpu.async_copy(tbl_s.at[src_v.at[b]], bufs[b], sems[b])

        def halfstep(r, b):
            buf, sem = bufs[b], sems[b]
            pltpu.make_async_copy(tbl_s.at[src_v.at[r]], buf, sem).wait()
            pltpu.sync_copy(buf, acc.at[dst_v.at[r]], add=True)

            @pl.when(r + 2 < cpw)
            def _():
                pltpu.async_copy(tbl_s.at[src_v.at[r + 2]], buf, sem)

        def body(g, carry):
            halfstep(2 * g, 0)
            halfstep(2 * g + 1, 1)
            return carry

        lax.fori_loop(0, cpw // 2, body, 0)
        if cpw % 2:
            halfstep(cpw - 1, 0)

        plsc.subcore_barrier()
        pltpu.sync_copy(acc.at[pl.ds(r0, rpt)], out.at[c, pl.ds(r0, rpt)])

    return agg


@functools.lru_cache(maxsize=None)
def _make_deg(n_pad: int, cpw: int):
    rpt = n_pad // NS
    mesh = plsc.VectorSubcoreMesh(core_axis_name="c", subcore_axis_name="s")

    @functools.partial(
        pl.kernel,
        out_type=jax.ShapeDtypeStruct((NC, n_pad, DEG_D), jnp.float32),
        mesh=mesh,
        scratch_types=[
            pltpu.VMEM((cpw, CHUNK), jnp.int32),
            pltpu.VMEM((CHUNK, DEG_D), jnp.float32),
            pltpu.VMEM_SHARED((n_pad, DEG_D), jnp.float32),
        ],
        compiler_params=pltpu.CompilerParams(use_tc_tiling_on_sc=False),
    )
    def deg(ones_hbm, zeros_hbm, dsts, out, dst_v, ones_v, acc):
        c = lax.axis_index("c")
        s = lax.axis_index("s")
        w = c * NS + s
        pltpu.sync_copy(dsts.at[pl.ds(w * cpw, cpw)], dst_v)
        pltpu.sync_copy(ones_hbm, ones_v)
        r0 = s * rpt
        pltpu.sync_copy(zeros_hbm, acc.at[pl.ds(r0, rpt)])
        plsc.subcore_barrier()

        def body(r, carry):
            pltpu.sync_copy(ones_v, acc.at[dst_v.at[r]], add=True)
            return carry

        lax.fori_loop(0, cpw, body, 0)

        plsc.subcore_barrier()
        pltpu.sync_copy(acc.at[pl.ds(r0, rpt)], out.at[c, pl.ds(r0, rpt)])

    return deg


def _mm1_body(x_ref, w_ref, d0_ref, d1_ref, h_ref, dinv_ref):
    deg = d0_ref[...] + d1_ref[...] + 1.0
    dinv = lax.rsqrt(deg)
    dinv_ref[...] = dinv
    h_ref[...] = jnp.dot(x_ref[...], w_ref[...],
                         preferred_element_type=jnp.float32) * dinv


def _mm2_body(a0_ref, a1_ref, hs_ref, dinv_ref, b_ref, w_ref, out_ref):
    dinv = dinv_ref[...]
    h = dinv * (a0_ref[...] + a1_ref[...] - hs_ref[...]) + b_ref[...]
    h = jnp.maximum(h, 0.0)
    out_ref[...] = jnp.dot(h, w_ref[...],
                           preferred_element_type=jnp.float32) * dinv


def _mm3_body(a0_ref, a1_ref, hs_ref, dinv_ref, b_ref, fcw_ref, fcb_ref,
              emb_ref, log_ref):
    dinv = dinv_ref[...]
    h = dinv * (a0_ref[...] + a1_ref[...] - hs_ref[...]) + b_ref[...]
    h = jnp.maximum(h, 0.0)
    emb_ref[...] = h
    log_ref[...] = jnp.dot(h, fcw_ref[...],
                           preferred_element_type=jnp.float32) + fcb_ref[...]


def _row_spec(d):
    return pl.BlockSpec((BLK, d), lambda i: (i, 0))


def _full_spec(shape):
    return pl.BlockSpec(shape, lambda i: (0,) * len(shape))


def kernel(x, edge_index, W1, b1, W2, b2, fcW, fcb):
    n, in_dim = x.shape
    h1d = W1.shape[1]
    h2d = W2.shape[1]
    od = fcW.shape[1]

    src = edge_index[0].astype(jnp.int32)
    dst = edge_index[1].astype(jnp.int32)
    e = src.shape[0]

    n_pad = ((n + 1 + BLK - 1) // BLK) * BLK
    cpw = -(-(-(-e // CHUNK)) // NW)
    cpw = ((cpw + 7) // 8) * 8
    e_pad = cpw * NW * CHUNK

    src_p = jnp.concatenate(
        [src, jnp.zeros((e_pad - e,), jnp.int32)]).reshape(-1, CHUNK)
    dst_p = jnp.concatenate(
        [dst, jnp.full((e_pad - e,), n, jnp.int32)]).reshape(-1, CHUNK)
    x_p = jnp.pad(x, ((0, n_pad - n), (0, 0)))

    ones_rows = jnp.ones((CHUNK, DEG_D), jnp.float32)
    zeros_stripe = jnp.zeros((n_pad // NS, DEG_D), jnp.float32)

    deg_out = _make_deg(n_pad, cpw)(ones_rows, zeros_stripe, dst_p)
    d0 = deg_out[0, :, 0:1]
    d1 = deg_out[1, :, 0:1]

    grid = (n_pad // BLK,)

    h1s, dinv = pl.pallas_call(
        _mm1_body,
        grid=grid,
        in_specs=[_row_spec(in_dim), _full_spec((in_dim, h1d)),
                  _row_spec(1), _row_spec(1)],
        out_specs=[_row_spec(h1d), _row_spec(1)],
        out_shape=[jax.ShapeDtypeStruct((n_pad, h1d), jnp.float32),
                   jax.ShapeDtypeStruct((n_pad, 1), jnp.float32)],
    )(x_p, W1, d0, d1)

    agg1 = _make_agg(n_pad, h1d, cpw)(h1s, src_p, dst_p)

    h2s = pl.pallas_call(
        _mm2_body,
        grid=grid,
        in_specs=[_row_spec(h1d), _row_spec(h1d), _row_spec(h1d), _row_spec(1),
                  _full_spec((1, h1d)), _full_spec((h1d, h2d))],
        out_specs=_row_spec(h2d),
        out_shape=jax.ShapeDtypeStruct((n_pad, h2d), jnp.float32),
    )(agg1[0], agg1[1], h1s, dinv, b1.reshape(1, -1), W2)

    agg2 = _make_agg(n_pad, h2d, cpw)(h2s, src_p, dst_p)

    emb, logits = pl.pallas_call(
        _mm3_body,
        grid=grid,
        in_specs=[_row_spec(h2d), _row_spec(h2d), _row_spec(h2d), _row_spec(1),
                  _full_spec((1, h2d)), _full_spec((h2d, od)),
                  _full_spec((1, od))],
        out_specs=[_row_spec(h2d), _row_spec(od)],
        out_shape=[jax.ShapeDtypeStruct((n_pad, h2d), jnp.float32),
                   jax.ShapeDtypeStruct((n_pad, od), jnp.float32)],
    )(agg2[0], agg2[1], h2s, dinv, b2.reshape(1, -1), fcW, fcb.reshape(1, -1))

    return emb[:n], logits[:n]
